# Initial kernel scaffold; baseline (speedup 1.0000x reference)
import jax
import jax.numpy as jnp
from jax.experimental import pallas as pl


def kernel(h, gate_w, experts_gate_up, experts_down, sh_gate_w, sh_up_w, sh_down_w, se_gate_w):
    raise NotImplementedError("write your pallas kernel here")


# speedup vs baseline: 3.8302x; 3.8302x over previous
import functools

import jax
import jax.numpy as jnp
from jax.experimental import pallas as pl
from jax.experimental.pallas import tpu as pltpu

B, T, H = 8, 1, 2048
E, K, I, SI = 64, 8, 512, 512
N = B * T
P = N * K


def _router_shared_kernel(h_ref, gate_w_ref, sh_gate_ref, sh_up_ref,
                          sh_down_ref, se_gate_ref,
                          sh_out_ref, ids_ref, wts_ref):
    hv = h_ref[:]

    logits = jax.lax.dot_general(
        hv, gate_w_ref[:], (((1,), (1,)), ((), ())),
        preferred_element_type=jnp.float32)
    col = jax.lax.broadcasted_iota(jnp.int32, (N, E), 1)
    masked = logits
    vals = []
    neg_inf = jnp.float32(-jnp.inf)
    for k in range(K):
        m = jnp.max(masked, axis=1, keepdims=True)
        is_m = masked == m
        idx = jnp.min(jnp.where(is_m, col, E), axis=1, keepdims=True)
        ids_ref[:, k] = idx[:, 0]
        vals.append(m)
        masked = jnp.where(col == idx, neg_inf, masked)
    topv = jnp.concatenate(vals, axis=1)
    ex = jnp.exp(topv - topv[:, 0:1])
    wts_ref[:] = ex / jnp.sum(ex, axis=1, keepdims=True)

    g = jax.lax.dot_general(hv, sh_gate_ref[:], (((1,), (1,)), ((), ())),
                            preferred_element_type=jnp.float32)
    u = jax.lax.dot_general(hv, sh_up_ref[:], (((1,), (1,)), ((), ())),
                            preferred_element_type=jnp.float32)
    inter = g * jax.nn.sigmoid(g) * u
    so = jax.lax.dot_general(inter, sh_down_ref[:], (((1,), (1,)), ((), ())),
                             preferred_element_type=jnp.float32)
    se = jax.nn.sigmoid(
        jax.lax.dot_general(hv, se_gate_ref[:], (((1,), (1,)), ((), ())),
                            preferred_element_type=jnp.float32))
    sh_out_ref[:] = se * so


def _moe_ffn_kernel(ids_ref, tok_ref, wts_ref,
                    h_ref, gu_ref, dn_ref, sh_ref, out_ref):
    p = pl.program_id(0)

    @pl.when(p == 0)
    def _():
        out_ref[:] = sh_ref[:]

    hv = h_ref[0]
    gup = jax.lax.dot_general(hv, gu_ref[0], (((1,), (1,)), ((), ())),
                              preferred_element_type=jnp.float32)
    gate = gup[:, :I]
    up = gup[:, I:]
    inter = gate * jax.nn.sigmoid(gate) * up
    so = jax.lax.dot_general(inter, dn_ref[0], (((1,), (1,)), ((), ())),
                             preferred_element_type=jnp.float32)
    w = wts_ref[p]
    t = tok_ref[p]
    row = jax.lax.broadcasted_iota(jnp.int32, (N, 1), 0)
    out_ref[:] += jnp.where(row == t, w * so, jnp.float32(0.0))


@functools.partial(jax.jit, static_argnames=())
def _run(h, gate_w, experts_gate_up, experts_down, sh_gate_w, sh_up_w,
         sh_down_w, se_gate_w):
    h_flat = h.reshape(N, H)

    sh_out, ids, wts = pl.pallas_call(
        _router_shared_kernel,
        out_shape=(
            jax.ShapeDtypeStruct((N, H), jnp.float32),
            jax.ShapeDtypeStruct((N, K), jnp.int32),
            jax.ShapeDtypeStruct((N, K), jnp.float32),
        ),
    )(h_flat, gate_w, sh_gate_w, sh_up_w, sh_down_w, se_gate_w)

    ids_flat = ids.reshape(P)
    order = jnp.argsort(ids_flat)
    ids_s = ids_flat[order]
    wts_s = wts.reshape(P)[order]
    tok_s = (order // K).astype(jnp.int32)

    grid_spec = pltpu.PrefetchScalarGridSpec(
        num_scalar_prefetch=3,
        grid=(P,),
        in_specs=[
            pl.BlockSpec((1, 1, H), lambda p, ids, tok, w: (tok[p], 0, 0)),
            pl.BlockSpec((1, 2 * I, H), lambda p, ids, tok, w: (ids[p], 0, 0)),
            pl.BlockSpec((1, H, I), lambda p, ids, tok, w: (ids[p], 0, 0)),
            pl.BlockSpec((N, H), lambda p, ids, tok, w: (0, 0)),
        ],
        out_specs=pl.BlockSpec((N, H), lambda p, ids, tok, w: (0, 0)),
    )
    out = pl.pallas_call(
        _moe_ffn_kernel,
        grid_spec=grid_spec,
        out_shape=jax.ShapeDtypeStruct((N, H), jnp.float32),
        compiler_params=pltpu.CompilerParams(
            dimension_semantics=("arbitrary",)),
    )(ids_s, tok_s, wts_s, h_flat.reshape(N, 1, H), experts_gate_up,
      experts_down, sh_out)

    return out.reshape(B, T, H)


def kernel(h, gate_w, experts_gate_up, experts_down, sh_gate_w, sh_up_w,
           sh_down_w, se_gate_w):
    return _run(h, gate_w, experts_gate_up, experts_down, sh_gate_w,
                sh_up_w, sh_down_w, se_gate_w)
